# baseline (speedup 1.0000x reference)
import functools

import jax
import jax.numpy as jnp
from jax import lax
from jax.experimental import pallas as pl
from jax.experimental.pallas import tpu as pltpu
from jax.experimental.pallas import tpu_sc as plsc

_INFO = plsc.get_sparse_core_info()
_NC = _INFO.num_cores
_NS = _INFO.num_subcores
_NW = _NC * _NS

_LANES = 16
_SGROUPS = 4
_BSLABS = _NW // _SGROUPS


def _make_lookup(seq, batch, n_vocab, d):
    assert batch % (_BSLABS * _LANES) == 0 and seq % _SGROUPS == 0
    b_slab = batch // _BSLABS
    s_per_w = seq // _SGROUPS
    n_groups = b_slab // _LANES

    mesh = plsc.VectorSubcoreMesh(core_axis_name="c", subcore_axis_name="s")

    @functools.partial(
        pl.kernel,
        out_type=jax.ShapeDtypeStruct((seq, d, batch), jnp.float32),
        mesh=mesh,
        scratch_types=[
            pltpu.VMEM((n_vocab * d,), jnp.float32),
            pltpu.VMEM((b_slab,), jnp.int32),
            pltpu.VMEM((d, b_slab), jnp.float32),
        ],
        compiler_params=pltpu.CompilerParams(needs_layout_passes=False),
    )
    def lookup_kernel(idx_hbm, table_hbm, out_hbm, table_v, idx_v, blk_v):
        wid = lax.axis_index("s") * _NC + lax.axis_index("c")
        sg = wid // _BSLABS
        bs = wid % _BSLABS
        s0 = sg * s_per_w
        b0 = bs * b_slab

        pltpu.sync_copy(table_hbm, table_v)

        def do_s(i, carry):
            s = s0 + i
            pltpu.sync_copy(idx_hbm.at[pl.ds(s * batch + b0, b_slab)], idx_v)

            def do_group(g, c2):
                idxv = idx_v[pl.ds(g * _LANES, _LANES)]
                scaled = idxv * d
                for dd in range(d):
                    vals = plsc.load_gather(table_v, [scaled + dd])
                    blk_v[dd, pl.ds(g * _LANES, _LANES)] = vals
                return c2

            lax.fori_loop(0, n_groups, do_group, 0)
            pltpu.sync_copy(blk_v, out_hbm.at[s, :, pl.ds(b0, b_slab)])
            return carry

        lax.fori_loop(0, s_per_w, do_s, 0)

    return lookup_kernel


def kernel(x, table):
    b, s = x.shape
    v, d = table.shape
    xt = x.T.reshape(s * b).astype(jnp.int32)
    outt = _make_lookup(s, b, v, d)(xt, table.reshape(v * d))
    return outt.transpose(2, 0, 1)


# speedup vs baseline: 1.6022x; 1.6022x over previous
import functools

import jax
import jax.numpy as jnp
from jax import lax
from jax.experimental import pallas as pl
from jax.experimental.pallas import tpu as pltpu
from jax.experimental.pallas import tpu_sc as plsc

_INFO = plsc.get_sparse_core_info()
_NC = _INFO.num_cores
_NS = _INFO.num_subcores
_NW = _NC * _NS

_LANES = 16
_SGROUPS = 4
_BSLABS = _NW // _SGROUPS


def _make_lookup(seq, batch, n_vocab, d):
    assert batch % (_BSLABS * _LANES) == 0 and seq % _SGROUPS == 0
    b_slab = batch // _BSLABS
    s_per_w = seq // _SGROUPS
    n_groups = b_slab // _LANES

    mesh = plsc.VectorSubcoreMesh(core_axis_name="c", subcore_axis_name="s")

    @functools.partial(
        pl.kernel,
        out_type=jax.ShapeDtypeStruct((seq, d, batch), jnp.float32),
        mesh=mesh,
        scratch_types=[
            pltpu.VMEM((n_vocab * d,), jnp.float32),
            pltpu.VMEM((b_slab,), jnp.int32),
            pltpu.VMEM((d, b_slab), jnp.float32),
        ],
        compiler_params=pltpu.CompilerParams(needs_layout_passes=False),
    )
    def lookup_kernel(idx_hbm, table_hbm, out_hbm, table_v, idx_v, blk_v):
        wid = lax.axis_index("s") * _NC + lax.axis_index("c")
        sg = wid // _BSLABS
        bs = wid % _BSLABS
        s0 = sg * s_per_w
        b0 = bs * b_slab

        pltpu.sync_copy(table_hbm, table_v)

        def do_s(i, carry):
            s = s0 + i
            pltpu.sync_copy(idx_hbm.at[pl.ds(s * batch + b0, b_slab)], idx_v)

            def do_group(g, c2):
                idxv = idx_v[pl.ds(g * _LANES, _LANES)]
                scaled = idxv * d
                for dd0 in range(0, d, 8):
                    vals = [plsc.load_gather(table_v, [scaled + (dd0 + j)])
                            for j in range(8)]
                    for j in range(8):
                        blk_v[dd0 + j, pl.ds(g * _LANES, _LANES)] = vals[j]
                return c2

            lax.fori_loop(0, n_groups, do_group, 0)
            pltpu.sync_copy(blk_v, out_hbm.at[s, :, pl.ds(b0, b_slab)])
            return carry

        lax.fori_loop(0, s_per_w, do_s, 0)

    return lookup_kernel


def kernel(x, table):
    b, s = x.shape
    v, d = table.shape
    xt = x.T.reshape(s * b).astype(jnp.int32)
    outt = _make_lookup(s, b, v, d)(xt, table.reshape(v * d))
    return outt.transpose(2, 0, 1)


# speedup vs baseline: 5.5001x; 3.4330x over previous
import functools

import jax
import jax.numpy as jnp
from jax import lax
from jax.experimental import pallas as pl
from jax.experimental.pallas import tpu as pltpu
from jax.experimental.pallas import tpu_sc as plsc

_INFO = plsc.get_sparse_core_info()
_NC = _INFO.num_cores
_NS = _INFO.num_subcores
_NW = _NC * _NS

_LANES = 16
_SGROUPS = 4
_BSLABS = _NW // _SGROUPS


def _make_lookup(seq, batch, n_vocab, d):
    assert batch % (_BSLABS * _LANES) == 0 and seq % _SGROUPS == 0
    b_slab = batch // _BSLABS
    s_per_w = seq // _SGROUPS
    n_groups = b_slab // _LANES

    mesh = plsc.VectorSubcoreMesh(core_axis_name="c", subcore_axis_name="s")

    @functools.partial(
        pl.kernel,
        out_type=jax.ShapeDtypeStruct((seq, d, batch), jnp.float32),
        mesh=mesh,
        scratch_types=[
            pltpu.VMEM((n_vocab * (d + 1),), jnp.float32),
            pltpu.VMEM((b_slab,), jnp.int32),
            pltpu.VMEM((d, b_slab), jnp.float32),
        ],
        compiler_params=pltpu.CompilerParams(needs_layout_passes=False),
    )
    def lookup_kernel(idx_hbm, table_hbm, out_hbm, table_v, idx_v, blk_v):
        wid = lax.axis_index("s") * _NC + lax.axis_index("c")
        sg = wid // _BSLABS
        bs = wid % _BSLABS
        s0 = sg * s_per_w
        b0 = bs * b_slab

        pltpu.sync_copy(table_hbm, table_v)

        def do_s(i, carry):
            s = s0 + i
            pltpu.sync_copy(idx_hbm.at[pl.ds(s * batch + b0, b_slab)], idx_v)

            def do_group(g, c2):
                idxv = idx_v[pl.ds(g * _LANES, _LANES)]
                scaled = idxv * (d + 1)
                for dd0 in range(0, d, 8):
                    vals = [plsc.load_gather(table_v, [scaled + (dd0 + j)])
                            for j in range(8)]
                    for j in range(8):
                        blk_v[dd0 + j, pl.ds(g * _LANES, _LANES)] = vals[j]
                return c2

            lax.fori_loop(0, n_groups, do_group, 0)
            pltpu.sync_copy(blk_v, out_hbm.at[s, :, pl.ds(b0, b_slab)])
            return carry

        lax.fori_loop(0, s_per_w, do_s, 0)

    return lookup_kernel


def kernel(x, table):
    b, s = x.shape
    v, d = table.shape
    xt = x.T.reshape(s * b).astype(jnp.int32)
    tablep = jnp.pad(table, ((0, 0), (0, 1))).reshape(v * (d + 1))
    outt = _make_lookup(s, b, v, d)(xt, tablep)
    return outt.transpose(2, 0, 1)


# speedup vs baseline: 8.7812x; 1.5966x over previous
import functools

import jax
import jax.numpy as jnp
from jax import lax
from jax.experimental import pallas as pl
from jax.experimental.pallas import tpu as pltpu
from jax.experimental.pallas import tpu_sc as plsc

_INFO = plsc.get_sparse_core_info()
_NC = _INFO.num_cores
_NS = _INFO.num_subcores
_NW = _NC * _NS

_LANES = 16
_SGROUPS = 4
_BSLABS = _NW // _SGROUPS


def _make_lookup(seq, batch, n_vocab, d):
    assert batch % (_BSLABS * _LANES) == 0 and seq % _SGROUPS == 0
    b_slab = batch // _BSLABS
    s_per_w = seq // _SGROUPS
    n_groups = b_slab // _LANES
    n_pairs = s_per_w // 2
    assert s_per_w % 2 == 0 and n_pairs >= 3

    mesh = plsc.VectorSubcoreMesh(core_axis_name="c", subcore_axis_name="s")

    @functools.partial(
        pl.kernel,
        out_type=jax.ShapeDtypeStruct((seq, d, batch), jnp.float32),
        mesh=mesh,
        scratch_types=[
            pltpu.VMEM((n_vocab * (d + 1),), jnp.float32),
            pltpu.VMEM((2, b_slab), jnp.int32),
            pltpu.VMEM((2, d, b_slab), jnp.float32),
            pltpu.SemaphoreType.DMA,
            pltpu.SemaphoreType.DMA,
            pltpu.SemaphoreType.DMA,
            pltpu.SemaphoreType.DMA,
        ],
        compiler_params=pltpu.CompilerParams(needs_layout_passes=False),
    )
    def lookup_kernel(idx_hbm, table_hbm, out_hbm, table_v, idx_v, blk_v,
                      i0, i1, o0, o1):
        wid = lax.axis_index("s") * _NC + lax.axis_index("c")
        sg = wid // _BSLABS
        bs = wid % _BSLABS
        s0 = sg * s_per_w
        b0 = bs * b_slab
        isem = (i0, i1)
        osem = (o0, o1)

        pltpu.sync_copy(table_hbm, table_v)

        def fire_idx(c, k):
            pltpu.async_copy(
                idx_hbm.at[pl.ds((s0 + c) * batch + b0, b_slab)],
                idx_v.at[k], isem[k])

        def wait_idx(k):
            pltpu.make_async_copy(idx_hbm.at[pl.ds(b0, b_slab)],
                                  idx_v.at[k], isem[k]).wait()

        def fire_out(c, k):
            pltpu.async_copy(blk_v.at[k],
                             out_hbm.at[s0 + c, :, pl.ds(b0, b_slab)],
                             osem[k])

        def wait_out(k):
            pltpu.make_async_copy(blk_v.at[k],
                                  out_hbm.at[s0, :, pl.ds(b0, b_slab)],
                                  osem[k]).wait()

        def compute(k):
            def do_group(g, c2):
                idxv = idx_v[k, pl.ds(g * _LANES, _LANES)]
                scaled = idxv * (d + 1)
                for dd0 in range(0, d, 8):
                    vals = [plsc.load_gather(table_v, [scaled + (dd0 + j)])
                            for j in range(8)]
                    for j in range(8):
                        blk_v[k, dd0 + j, pl.ds(g * _LANES, _LANES)] = vals[j]
                return c2

            lax.fori_loop(0, n_groups, do_group, 0)

        def chunk(c, k, fire_next, wait_o):
            wait_idx(k)
            if fire_next:
                fire_idx(c + 1, 1 - k)
            if wait_o:
                wait_out(k)
            compute(k)
            fire_out(c, k)

        fire_idx(0, 0)
        chunk(0, 0, True, False)
        chunk(1, 1, True, False)

        def body(p, carry):
            a = 2 * p
            chunk(a, 0, True, True)
            chunk(a + 1, 1, True, True)
            return carry

        lax.fori_loop(1, n_pairs - 1, body, 0)

        last = s_per_w - 2
        chunk(last, 0, True, True)
        chunk(last + 1, 1, False, True)
        wait_out(0)
        wait_out(1)

    return lookup_kernel


def kernel(x, table):
    b, s = x.shape
    v, d = table.shape
    xt = x.T.reshape(s * b).astype(jnp.int32)
    tablep = jnp.pad(table, ((0, 0), (0, 1))).reshape(v * (d + 1))
    outt = _make_lookup(s, b, v, d)(xt, tablep)
    return outt.transpose(2, 0, 1)


# speedup vs baseline: 12.2765x; 1.3980x over previous
import functools

import jax
import jax.numpy as jnp
from jax import lax
from jax.experimental import pallas as pl
from jax.experimental.pallas import tpu as pltpu
from jax.experimental.pallas import tpu_sc as plsc

_INFO = plsc.get_sparse_core_info()
_NC = _INFO.num_cores
_NS = _INFO.num_subcores
_NW = _NC * _NS

_LANES = 16
_SGROUPS = 4
_BSLABS = _NW // _SGROUPS


def _make_lookup(seq, batch, n_vocab, d):
    assert batch % (_BSLABS * _LANES) == 0 and seq % _SGROUPS == 0
    b_slab = batch // _BSLABS
    s_per_w = seq // _SGROUPS
    n_groups = b_slab // _LANES
    n_pairs = s_per_w // 2
    assert s_per_w % 2 == 0 and n_pairs >= 3

    mesh = plsc.VectorSubcoreMesh(core_axis_name="c", subcore_axis_name="s")

    @functools.partial(
        pl.kernel,
        out_type=jax.ShapeDtypeStruct((seq, d, batch), jnp.float32),
        mesh=mesh,
        scratch_types=[
            pltpu.VMEM((n_vocab * (d + 1),), jnp.float32),
            pltpu.VMEM((2, b_slab), jnp.int32),
            pltpu.VMEM((2, d, b_slab), jnp.float32),
            pltpu.SemaphoreType.DMA,
            pltpu.SemaphoreType.DMA,
            pltpu.SemaphoreType.DMA,
            pltpu.SemaphoreType.DMA,
        ],
        compiler_params=pltpu.CompilerParams(needs_layout_passes=False),
    )
    def lookup_kernel(idx_hbm, table_hbm, out_hbm, table_v, idx_v, blk_v,
                      i0, i1, o0, o1):
        wid = lax.axis_index("s") * _NC + lax.axis_index("c")
        sg = wid // _BSLABS
        bs = wid % _BSLABS
        s0 = sg * s_per_w
        b0 = bs * b_slab
        isem = (i0, i1)
        osem = (o0, o1)

        pltpu.sync_copy(table_hbm, table_v)

        def fire_idx(c, k):
            pltpu.async_copy(
                idx_hbm.at[pl.ds((s0 + c) * batch + b0, b_slab)],
                idx_v.at[k], isem[k])

        def wait_idx(k):
            pltpu.make_async_copy(idx_hbm.at[pl.ds(b0, b_slab)],
                                  idx_v.at[k], isem[k]).wait()

        def fire_out(c, k):
            pltpu.async_copy(blk_v.at[k],
                             out_hbm.at[s0 + c, :, pl.ds(b0, b_slab)],
                             osem[k])

        def wait_out(k):
            pltpu.make_async_copy(blk_v.at[k],
                                  out_hbm.at[s0, :, pl.ds(b0, b_slab)],
                                  osem[k]).wait()

        def compute(k):
            def do_group(g, c2):
                idxv = idx_v[k, pl.ds(g * _LANES, _LANES)]
                scaled = idxv * (d + 1)
                nb = d // 8
                prev = [plsc.load_gather(table_v, [scaled + j])
                        for j in range(8)]
                for m in range(1, nb):
                    cur = []
                    for j in range(8):
                        cur.append(
                            plsc.load_gather(table_v, [scaled + (m * 8 + j)]))
                        blk_v[k, (m - 1) * 8 + j,
                              pl.ds(g * _LANES, _LANES)] = prev[j]
                    prev = cur
                for j in range(8):
                    blk_v[k, d - 8 + j, pl.ds(g * _LANES, _LANES)] = prev[j]
                return c2

            lax.fori_loop(0, n_groups, do_group, 0)

        def chunk(c, k, fire_next, wait_o):
            wait_idx(k)
            if fire_next:
                fire_idx(c + 1, 1 - k)
            if wait_o:
                wait_out(k)
            compute(k)
            fire_out(c, k)

        fire_idx(0, 0)
        chunk(0, 0, True, False)
        chunk(1, 1, True, False)

        def body(p, carry):
            a = 2 * p
            chunk(a, 0, True, True)
            chunk(a + 1, 1, True, True)
            return carry

        lax.fori_loop(1, n_pairs - 1, body, 0)

        last = s_per_w - 2
        chunk(last, 0, True, True)
        chunk(last + 1, 1, False, True)
        wait_out(0)
        wait_out(1)

    return lookup_kernel


def kernel(x, table):
    b, s = x.shape
    v, d = table.shape
    xt = x.T.reshape(s * b).astype(jnp.int32)
    tablep = jnp.pad(table, ((0, 0), (0, 1))).reshape(v * (d + 1))
    outt = _make_lookup(s, b, v, d)(xt, tablep)
    return outt.transpose(2, 0, 1)
